# baseline (speedup 1.0000x reference)
import functools

import jax
import jax.numpy as jnp
import numpy as np
from jax import lax
from jax.experimental import pallas as pl
from jax.experimental.pallas import tpu as pltpu
from jax.experimental.pallas import tpu_sc as plsc

NC = 2
NS = 16
NW = NC * NS

D = 128
CHUNK = 128
HALF = 512

_COL_L = np.array([32 * (w // 16) + (w % 16) for w in range(64)], np.int32)
_COL_H = _COL_L + 16


def _make_gather_packed(n_rows: int):
    assert n_rows % (NW * CHUNK) == 0
    chunks_per_w = n_rows // (NW * CHUNK)
    rows_per_w = chunks_per_w * CHUNK
    assert chunks_per_w % 2 == 0

    mesh = plsc.VectorSubcoreMesh(
        core_axis_name="c", subcore_axis_name="s", num_cores=NC, num_subcores=NS
    )

    pairs_per_w = chunks_per_w // 2

    @functools.partial(
        pl.kernel,
        out_type=jax.ShapeDtypeStruct((n_rows // 2, D), jnp.uint32),
        mesh=mesh,
        compiler_params=pltpu.CompilerParams(needs_layout_passes=False),
        scratch_types=[
            pltpu.VMEM((chunks_per_w, CHUNK), jnp.int32),
            pltpu.VMEM((CHUNK, D), jnp.float32),
            pltpu.VMEM((CHUNK, D), jnp.float32),
            pltpu.VMEM((CHUNK, D), jnp.float32),
            pltpu.VMEM((CHUNK, D), jnp.float32),
            pltpu.VMEM((CHUNK, D), jnp.uint32),
            pltpu.VMEM((CHUNK, D), jnp.uint32),
            pltpu.SemaphoreType.DMA,
            pltpu.SemaphoreType.DMA,
            pltpu.SemaphoreType.DMA,
            pltpu.SemaphoreType.DMA,
            pltpu.SemaphoreType.DMA,
            pltpu.SemaphoreType.DMA,
        ],
    )
    def gather(idx_hbm, table_hbm, out_hbm, idx_v,
               ra0, ra1, rb0, rb1, pka, pkb,
               sga0, sga1, sgb0, sgb1, ssa, ssb):
        wid = lax.axis_index("s") * NC + lax.axis_index("c")
        base = wid * rows_per_w
        pltpu.sync_copy(idx_hbm.at[wid], idx_v)

        sets = ((ra0, ra1, pka, sga0, sga1, ssa),
                (rb0, rb1, pkb, sgb0, sgb1, ssb))
        hi_mask = jnp.uint32(0xFFFF0000)

        def start_pair(q, bs):
            r0, r1, _, sg0, sg1, _ = sets[bs]
            pltpu.async_copy(table_hbm.at[idx_v.at[2 * q]], r0, sg0)
            pltpu.async_copy(table_hbm.at[idx_v.at[2 * q + 1]], r1, sg1)

        def wait_pair(bs):
            r0, r1, _, sg0, sg1, _ = sets[bs]
            pltpu.make_async_copy(table_hbm.at[idx_v.at[0]], r0, sg0).wait()
            pltpu.make_async_copy(table_hbm.at[idx_v.at[0]], r1, sg1).wait()

        def convert(bs):
            r0, r1, pk, _, _, _ = sets[bs]

            def rowfn(r, carry):
                for g in range(D // 32):
                    a = plsc.bitcast(r0[r, pl.ds(32 * g, 16)], jnp.uint32)
                    c = plsc.bitcast(r0[r, pl.ds(32 * g + 16, 16)], jnp.uint32)
                    pk[r, pl.ds(16 * g, 16)] = (a >> 16) | (c & hi_mask)
                    a = plsc.bitcast(r1[r, pl.ds(32 * g, 16)], jnp.uint32)
                    c = plsc.bitcast(r1[r, pl.ds(32 * g + 16, 16)], jnp.uint32)
                    pk[r, pl.ds(64 + 16 * g, 16)] = (a >> 16) | (c & hi_mask)
                return carry

            lax.fori_loop(0, CHUNK, rowfn, 0)

        def start_store(q, bs):
            _, _, pk, _, _, ss = sets[bs]
            srow = pl.multiple_of((base // 2) + q * CHUNK, CHUNK)
            pltpu.async_copy(pk, out_hbm.at[pl.ds(srow, CHUNK)], ss)

        def wait_store(bs):
            _, _, pk, _, _, ss = sets[bs]
            pltpu.make_async_copy(pk, out_hbm.at[pl.ds(0, CHUNK)], ss).wait()

        start_pair(0, 0)
        if pairs_per_w > 1:
            start_pair(1, 1)
        for q in range(pairs_per_w):
            bs = q % 2
            wait_pair(bs)
            if q >= 2:
                wait_store(bs)
            convert(bs)
            start_store(q, bs)
            if q + 2 < pairs_per_w:
                start_pair(q + 2, bs)
        wait_store(0)
        if pairs_per_w > 1:
            wait_store(1)

    return gather


P = 5
BN = 8192


def _mlp_body(x_ref, w1_ref, b1_ref, w2_ref, b2_ref, o_ref):
    xi = x_ref[...]
    lo = lax.bitcast_convert_type(xi << 16, jnp.float32).astype(jnp.bfloat16)
    hi = lax.bitcast_convert_type(xi & jnp.uint32(0xFFFF0000),
                                  jnp.float32).astype(jnp.bfloat16)
    xcat = jnp.concatenate([lo, hi], axis=1)
    hab = jnp.dot(xcat, w1_ref[...], preferred_element_type=jnp.float32)
    hab = jnp.maximum(hab + b1_ref[...], 0.0).astype(jnp.bfloat16)
    o_lo = (jnp.dot(hab[:, :D], w2_ref[...], preferred_element_type=jnp.float32)
            + b2_ref[...])
    o_hi = (jnp.dot(hab[:, D:], w2_ref[...], preferred_element_type=jnp.float32)
            + b2_ref[...])
    for m in range(BN // 256):
        o_ref[256 * m:256 * m + 128, :] = o_lo[128 * m:128 * m + 128, :]
        o_ref[256 * m + 128:256 * m + 256, :] = o_hi[128 * m:128 * m + 128, :]


def _mlp_part_block(acc_ref, x_ref, w1_ref, b1_ref, w2_ref, b2_ref, o_ref):
    _mlp_body(x_ref, w1_ref, b1_ref, w2_ref, b2_ref, o_ref)


def _mlp_first_block(x_ref, w1_ref, b1_ref, w2_ref, b2_ref, o_ref):
    _mlp_body(x_ref, w1_ref, b1_ref, w2_ref, b2_ref, o_ref)


def kernel(sequences_vec, input_masks, table, W1, b1, W2, b2):
    B, S = sequences_vec.shape
    n = B * S
    C = W2.shape[1]

    npart = n // P
    chunks_per_w = npart // (NW * CHUNK)
    idx4 = sequences_vec.astype(jnp.int32).reshape(P, NW, chunks_per_w, CHUNK)

    W1L = W1[jnp.asarray(_COL_L), :]
    W1H = W1[jnp.asarray(_COL_H), :]
    Z = jnp.zeros((64, D), W1.dtype)
    W1ab = jnp.concatenate([
        jnp.concatenate([W1L, Z], axis=1),
        jnp.concatenate([Z, W1L], axis=1),
        jnp.concatenate([W1H, Z], axis=1),
        jnp.concatenate([Z, W1H], axis=1),
    ], axis=0).astype(jnp.bfloat16)
    W2b = W2.astype(jnp.bfloat16)
    b1r = jnp.concatenate([b1, b1]).reshape(1, 2 * D)
    b2r = b2.reshape(1, C)

    sc_gather = _make_gather_packed(npart)
    parts = [sc_gather(idx4[p], table) for p in range(P)]

    spp = npart // BN
    weight_specs = [
        pl.BlockSpec((2 * D, 2 * D), lambda i: (0, 0)),
        pl.BlockSpec((1, 2 * D), lambda i: (0, 0)),
        pl.BlockSpec((D, C), lambda i: (0, 0)),
        pl.BlockSpec((1, C), lambda i: (0, 0)),
    ]
    x_spec = pl.BlockSpec((BN // 2, D), lambda i: (i, 0))

    out = pl.pallas_call(
        _mlp_first_block,
        grid=(spp,),
        in_specs=[x_spec] + weight_specs,
        out_specs=pl.BlockSpec((BN, C), lambda i: (i, 0)),
        out_shape=jax.ShapeDtypeStruct((n, C), jnp.float32),
    )(parts[0], W1ab, b1r, W2b, b2r)

    for p in range(1, P):
        out = pl.pallas_call(
            _mlp_part_block,
            grid=(spp,),
            in_specs=[pl.BlockSpec(memory_space=pl.ANY), x_spec] + weight_specs,
            out_specs=pl.BlockSpec((BN, C), lambda i, p=p: (p * spp + i, 0)),
            out_shape=jax.ShapeDtypeStruct((n, C), jnp.float32),
            input_output_aliases={0: 0},
        )(out, parts[p], W1ab, b1r, W2b, b2r)

    return out.reshape(B, S, C)


# speedup vs baseline: 1.0078x; 1.0078x over previous
import functools

import jax
import jax.numpy as jnp
import numpy as np
from jax import lax
from jax.experimental import pallas as pl
from jax.experimental.pallas import tpu as pltpu
from jax.experimental.pallas import tpu_sc as plsc

NC = 2
NS = 16
NW = NC * NS

D = 128
CHUNK = 128
HALF = 512

_COL_L = np.array([32 * (w // 16) + (w % 16) for w in range(64)], np.int32)
_COL_H = _COL_L + 16


def _make_gather_packed(n_rows: int):
    assert n_rows % (NW * CHUNK) == 0
    chunks_per_w = n_rows // (NW * CHUNK)
    rows_per_w = chunks_per_w * CHUNK
    assert chunks_per_w % 2 == 0

    mesh = plsc.VectorSubcoreMesh(
        core_axis_name="c", subcore_axis_name="s", num_cores=NC, num_subcores=NS
    )

    pairs_per_w = chunks_per_w // 2

    @functools.partial(
        pl.kernel,
        out_type=jax.ShapeDtypeStruct((n_rows // 2, D), jnp.uint32),
        mesh=mesh,
        compiler_params=pltpu.CompilerParams(needs_layout_passes=False),
        scratch_types=[
            pltpu.VMEM((chunks_per_w, CHUNK), jnp.int32),
            pltpu.VMEM((CHUNK, D), jnp.float32),
            pltpu.VMEM((CHUNK, D), jnp.float32),
            pltpu.VMEM((CHUNK, D), jnp.float32),
            pltpu.VMEM((CHUNK, D), jnp.float32),
            pltpu.VMEM((CHUNK, D), jnp.uint32),
            pltpu.VMEM((CHUNK, D), jnp.uint32),
            pltpu.SemaphoreType.DMA,
            pltpu.SemaphoreType.DMA,
            pltpu.SemaphoreType.DMA,
            pltpu.SemaphoreType.DMA,
            pltpu.SemaphoreType.DMA,
            pltpu.SemaphoreType.DMA,
        ],
    )
    def gather(idx_hbm, table_hbm, out_hbm, idx_v,
               ra0, ra1, rb0, rb1, pka, pkb,
               sga0, sga1, sgb0, sgb1, ssa, ssb):
        wid = lax.axis_index("s") * NC + lax.axis_index("c")
        base = wid * rows_per_w
        pltpu.sync_copy(idx_hbm.at[wid], idx_v)

        sets = ((ra0, ra1, pka, sga0, sga1, ssa),
                (rb0, rb1, pkb, sgb0, sgb1, ssb))
        hi_mask = jnp.uint32(0xFFFF0000)

        def start_pair(q, bs):
            r0, r1, _, sg0, sg1, _ = sets[bs]
            pltpu.async_copy(table_hbm.at[idx_v.at[2 * q]], r0, sg0)
            pltpu.async_copy(table_hbm.at[idx_v.at[2 * q + 1]], r1, sg1)

        def wait_pair(bs):
            r0, r1, _, sg0, sg1, _ = sets[bs]
            pltpu.make_async_copy(table_hbm.at[idx_v.at[0]], r0, sg0).wait()
            pltpu.make_async_copy(table_hbm.at[idx_v.at[0]], r1, sg1).wait()

        def convert(bs):
            r0, r1, pk, _, _, _ = sets[bs]

            def rowfn(r, carry):
                for g in range(D // 32):
                    a = plsc.bitcast(r0[r, pl.ds(32 * g, 16)], jnp.uint32)
                    c = plsc.bitcast(r0[r, pl.ds(32 * g + 16, 16)], jnp.uint32)
                    pk[r, pl.ds(16 * g, 16)] = (a >> 16) | (c & hi_mask)
                    a = plsc.bitcast(r1[r, pl.ds(32 * g, 16)], jnp.uint32)
                    c = plsc.bitcast(r1[r, pl.ds(32 * g + 16, 16)], jnp.uint32)
                    pk[r, pl.ds(64 + 16 * g, 16)] = (a >> 16) | (c & hi_mask)
                return carry

            lax.fori_loop(0, CHUNK, rowfn, 0)

        def start_store(q, bs):
            _, _, pk, _, _, ss = sets[bs]
            srow = pl.multiple_of((base // 2) + q * CHUNK, CHUNK)
            pltpu.async_copy(pk, out_hbm.at[pl.ds(srow, CHUNK)], ss)

        def wait_store(bs):
            _, _, pk, _, _, ss = sets[bs]
            pltpu.make_async_copy(pk, out_hbm.at[pl.ds(0, CHUNK)], ss).wait()

        start_pair(0, 0)
        if pairs_per_w > 1:
            start_pair(1, 1)
        for q in range(pairs_per_w):
            bs = q % 2
            wait_pair(bs)
            if q >= 2:
                wait_store(bs)
            convert(bs)
            start_store(q, bs)
            if q + 2 < pairs_per_w:
                start_pair(q + 2, bs)
        wait_store(0)
        if pairs_per_w > 1:
            wait_store(1)

    return gather


P = 5
BN = 4096


def _mlp_body(x_ref, w1_ref, b1_ref, w2_ref, b2_ref, o_ref):
    xi = x_ref[...]
    lo = lax.bitcast_convert_type(xi << 16, jnp.float32).astype(jnp.bfloat16)
    hi = lax.bitcast_convert_type(xi & jnp.uint32(0xFFFF0000),
                                  jnp.float32).astype(jnp.bfloat16)
    xcat = jnp.concatenate([lo, hi], axis=1)
    hab = jnp.dot(xcat, w1_ref[...], preferred_element_type=jnp.float32)
    hab = jnp.maximum(hab + b1_ref[...], 0.0).astype(jnp.bfloat16)
    o_lo = (jnp.dot(hab[:, :D], w2_ref[...], preferred_element_type=jnp.float32)
            + b2_ref[...])
    o_hi = (jnp.dot(hab[:, D:], w2_ref[...], preferred_element_type=jnp.float32)
            + b2_ref[...])
    for m in range(BN // 256):
        o_ref[256 * m:256 * m + 128, :] = o_lo[128 * m:128 * m + 128, :]
        o_ref[256 * m + 128:256 * m + 256, :] = o_hi[128 * m:128 * m + 128, :]


def _mlp_part_block(acc_ref, x_ref, w1_ref, b1_ref, w2_ref, b2_ref, o_ref):
    _mlp_body(x_ref, w1_ref, b1_ref, w2_ref, b2_ref, o_ref)


def _mlp_first_block(x_ref, w1_ref, b1_ref, w2_ref, b2_ref, o_ref):
    _mlp_body(x_ref, w1_ref, b1_ref, w2_ref, b2_ref, o_ref)


def kernel(sequences_vec, input_masks, table, W1, b1, W2, b2):
    B, S = sequences_vec.shape
    n = B * S
    C = W2.shape[1]

    npart = n // P
    chunks_per_w = npart // (NW * CHUNK)
    idx4 = sequences_vec.astype(jnp.int32).reshape(P, NW, chunks_per_w, CHUNK)

    W1L = W1[jnp.asarray(_COL_L), :]
    W1H = W1[jnp.asarray(_COL_H), :]
    Z = jnp.zeros((64, D), W1.dtype)
    W1ab = jnp.concatenate([
        jnp.concatenate([W1L, Z], axis=1),
        jnp.concatenate([Z, W1L], axis=1),
        jnp.concatenate([W1H, Z], axis=1),
        jnp.concatenate([Z, W1H], axis=1),
    ], axis=0).astype(jnp.bfloat16)
    W2b = W2.astype(jnp.bfloat16)
    b1r = jnp.concatenate([b1, b1]).reshape(1, 2 * D)
    b2r = b2.reshape(1, C)

    sc_gather = _make_gather_packed(npart)
    parts = [sc_gather(idx4[p], table) for p in range(P)]

    spp = npart // BN
    weight_specs = [
        pl.BlockSpec((2 * D, 2 * D), lambda i: (0, 0)),
        pl.BlockSpec((1, 2 * D), lambda i: (0, 0)),
        pl.BlockSpec((D, C), lambda i: (0, 0)),
        pl.BlockSpec((1, C), lambda i: (0, 0)),
    ]
    x_spec = pl.BlockSpec((BN // 2, D), lambda i: (i, 0))

    out = pl.pallas_call(
        _mlp_first_block,
        grid=(spp,),
        in_specs=[x_spec] + weight_specs,
        out_specs=pl.BlockSpec((BN, C), lambda i: (i, 0)),
        out_shape=jax.ShapeDtypeStruct((n, C), jnp.float32),
    )(parts[0], W1ab, b1r, W2b, b2r)

    for p in range(1, P):
        out = pl.pallas_call(
            _mlp_part_block,
            grid=(spp,),
            in_specs=[pl.BlockSpec(memory_space=pl.ANY), x_spec] + weight_specs,
            out_specs=pl.BlockSpec((BN, C), lambda i, p=p: (p * spp + i, 0)),
            out_shape=jax.ShapeDtypeStruct((n, C), jnp.float32),
            input_output_aliases={0: 0},
        )(out, parts[p], W1ab, b1r, W2b, b2r)

    return out.reshape(B, S, C)
